# baseline (speedup 1.0000x reference)
import jax
import jax.numpy as jnp
from jax.experimental import pallas as pl

E = 8
TOPK = 2
D_MODEL = 1024
D_FF = 2048
T = 1024
BF = 512
F = D_FF // BF


def _routing_body(g_ref, comb_ref):
    g = g_ref[...]
    lanes = jax.lax.broadcasted_iota(jnp.int32, (T, E), 1)
    m1 = jnp.max(g, axis=-1, keepdims=True)
    i1 = jnp.argmax(g, axis=-1)[:, None]
    g2 = jnp.where(lanes == i1, -jnp.inf, g)
    m2 = jnp.max(g2, axis=-1, keepdims=True)
    i2 = jnp.argmax(g2, axis=-1)[:, None]
    wa = 1.0 / (1.0 + jnp.exp(m2 - m1))
    wb = 1.0 - wa
    comb_ref[...] = jnp.where(lanes == i1, wa, 0.0) + jnp.where(lanes == i2, wb, 0.0)


def _moe_body(comb_ref, x_ref, w1g_ref, w1u_ref, w2_ref, out_ref):
    e = pl.program_id(0)
    f = pl.program_id(1)
    x = x_ref[...].astype(jnp.bfloat16)
    g = jnp.dot(x, w1g_ref[0].astype(jnp.bfloat16), preferred_element_type=jnp.float32)
    u = jnp.dot(x, w1u_ref[0].astype(jnp.bfloat16), preferred_element_type=jnp.float32)
    a = (g * jax.lax.logistic(g) * u).astype(jnp.bfloat16)
    y = jnp.dot(a, w2_ref[0].astype(jnp.bfloat16), preferred_element_type=jnp.float32)
    lanes = jax.lax.broadcasted_iota(jnp.int32, (T, E), 1)
    scale = jnp.sum(jnp.where(lanes == e, comb_ref[...], 0.0), axis=1, keepdims=True)
    contrib = y * scale

    @pl.when(jnp.logical_and(e == 0, f == 0))
    def _():
        out_ref[...] = jnp.zeros_like(out_ref)

    out_ref[...] += contrib


def kernel(x, gating_output, w1, w2):
    combine = pl.pallas_call(
        _routing_body,
        out_shape=jax.ShapeDtypeStruct((T, E), jnp.float32),
    )(gating_output)

    out = pl.pallas_call(
        _moe_body,
        grid=(E, F),
        in_specs=[
            pl.BlockSpec((T, E), lambda e, f: (0, 0)),
            pl.BlockSpec((T, D_MODEL), lambda e, f: (0, 0)),
            pl.BlockSpec((1, D_MODEL, BF), lambda e, f: (e, 0, f)),
            pl.BlockSpec((1, D_MODEL, BF), lambda e, f: (e, 0, F + f)),
            pl.BlockSpec((1, BF, D_MODEL), lambda e, f: (e, f, 0)),
        ],
        out_specs=pl.BlockSpec((T, D_MODEL), lambda e, f: (0, 0)),
        out_shape=jax.ShapeDtypeStruct((T, D_MODEL), jnp.float32),
    )(combine, x, w1, w1, w2)
    return out


# speedup vs baseline: 1.0753x; 1.0753x over previous
import jax
import jax.numpy as jnp
from jax.experimental import pallas as pl

E = 8
TOPK = 2
D_MODEL = 1024
D_FF = 2048
T = 1024
BF = 1024
F = D_FF // BF


def _routing_body(g_ref, comb_ref):
    g = g_ref[...]
    lanes = jax.lax.broadcasted_iota(jnp.int32, (T, E), 1)
    m1 = jnp.max(g, axis=-1, keepdims=True)
    i1 = jnp.argmax(g, axis=-1)[:, None]
    g2 = jnp.where(lanes == i1, -jnp.inf, g)
    m2 = jnp.max(g2, axis=-1, keepdims=True)
    i2 = jnp.argmax(g2, axis=-1)[:, None]
    wa = 1.0 / (1.0 + jnp.exp(m2 - m1))
    wb = 1.0 - wa
    comb_ref[...] = jnp.where(lanes == i1, wa, 0.0) + jnp.where(lanes == i2, wb, 0.0)


def _moe_body(comb_ref, x_ref, w1g_ref, w1u_ref, w2_ref, out_ref):
    e = pl.program_id(0)
    f = pl.program_id(1)
    x = x_ref[...]
    g = jnp.dot(x, w1g_ref[0], preferred_element_type=jnp.float32)
    u = jnp.dot(x, w1u_ref[0], preferred_element_type=jnp.float32)
    a = g * jax.lax.logistic(g) * u
    y = jnp.dot(a, w2_ref[0], preferred_element_type=jnp.float32)
    lanes = jax.lax.broadcasted_iota(jnp.int32, (T, E), 1)
    scale = jnp.sum(jnp.where(lanes == e, comb_ref[...], 0.0), axis=1, keepdims=True)
    contrib = y * scale

    @pl.when(jnp.logical_and(e == 0, f == 0))
    def _():
        out_ref[...] = jnp.zeros_like(out_ref)

    out_ref[...] += contrib


def kernel(x, gating_output, w1, w2):
    combine = pl.pallas_call(
        _routing_body,
        out_shape=jax.ShapeDtypeStruct((T, E), jnp.float32),
    )(gating_output)

    out = pl.pallas_call(
        _moe_body,
        grid=(E, F),
        in_specs=[
            pl.BlockSpec((T, E), lambda e, f: (0, 0)),
            pl.BlockSpec((T, D_MODEL), lambda e, f: (0, 0)),
            pl.BlockSpec((1, D_MODEL, BF), lambda e, f: (e, 0, f)),
            pl.BlockSpec((1, D_MODEL, BF), lambda e, f: (e, 0, F + f)),
            pl.BlockSpec((1, BF, D_MODEL), lambda e, f: (e, f, 0)),
        ],
        out_specs=pl.BlockSpec((T, D_MODEL), lambda e, f: (0, 0)),
        out_shape=jax.ShapeDtypeStruct((T, D_MODEL), jnp.float32),
    )(combine, x, w1, w1, w2)
    return out
